# baseline (speedup 1.0000x reference)
import functools

import jax
import jax.numpy as jnp
from jax import lax
from jax.experimental import pallas as pl
from jax.experimental.pallas import tpu as pltpu
from jax.experimental.pallas import tpu_sc as plsc

_NUM_DIM = 64
_PAD_DIM = 128
_BATCH = 4096
_SEQ = 200

_info = plsc.get_sparse_core_info()
_NC, _NS = _info.num_cores, _info.num_subcores
_NW = _NC * _NS
_TOTAL = _BATCH * _SEQ
_ROWS_PER_W = _TOTAL // _NW
_CH = 128
_NCHUNK = _ROWS_PER_W // _CH
_PE2 = _SEQ + _CH - 8
_D = 4
_HD = 32
_NG = _NCHUNK // _D


@functools.partial(
    pl.kernel,
    out_type=jax.ShapeDtypeStruct((_TOTAL, _PAD_DIM), jnp.float32),
    mesh=plsc.VectorSubcoreMesh(core_axis_name="c", subcore_axis_name="s"),
    scratch_types=[
        pltpu.VMEM_SHARED((_PE2, _NUM_DIM), jnp.float32),
        [pltpu.VMEM((_CH,), jnp.int32) for _ in range(_D)],
        [pltpu.VMEM((_CH, _HD), jnp.float32) for _ in range(_D)],
        [pltpu.VMEM((_CH, _HD), jnp.float32) for _ in range(_D)],
        [pltpu.SemaphoreType.DMA for _ in range(_D)],
        [pltpu.SemaphoreType.DMA for _ in range(_D)],
        [pltpu.SemaphoreType.DMA for _ in range(_D)],
    ],
    compiler_params=pltpu.CompilerParams(use_tc_tiling_on_sc=False),
)
def _embed_sc(x_hbm, pe_hbm, ta_hbm, tb_hbm, out_hbm,
              pe_sh, idxs, bufsa, bufsb, sem_i, sem_p, sem_g):
    wid = lax.axis_index("s") * _NC + lax.axis_index("c")
    base = wid * _ROWS_PER_W

    @pl.when(lax.axis_index("s") == 0)
    def _():
        pltpu.sync_copy(pe_hbm, pe_sh)

    plsc.subcore_barrier()

    def fire_inputs(j, c):
        off = base + c * _CH
        r = (c * _CH) % _SEQ
        pltpu.async_copy(x_hbm.at[pl.ds(off, _CH)], idxs[j], sem_i[j])
        pltpu.async_copy(pe_sh.at[pl.ds(r, _CH), pl.ds(0, _HD)],
                         bufsa[j], sem_p[j])
        pltpu.async_copy(pe_sh.at[pl.ds(r, _CH), pl.ds(_HD, _HD)],
                         bufsb[j], sem_p[j])

    def wait_inputs(j, c):
        off = base + c * _CH
        r = (c * _CH) % _SEQ
        pltpu.make_async_copy(x_hbm.at[pl.ds(off, _CH)], idxs[j],
                              sem_i[j]).wait()
        pltpu.make_async_copy(pe_sh.at[pl.ds(r, _CH), pl.ds(0, _HD)],
                              bufsa[j], sem_p[j]).wait()
        pltpu.make_async_copy(pe_sh.at[pl.ds(r, _CH), pl.ds(_HD, _HD)],
                              bufsb[j], sem_p[j]).wait()

    for j in range(_D):
        fire_inputs(j, j)

    def group(g, carry):
        for j in range(_D):
            c = g * _D + j
            wait_inputs(j, c)
            pltpu.async_copy(ta_hbm.at[idxs[j]], bufsa[j], sem_g[j],
                             add=True)
            pltpu.async_copy(tb_hbm.at[idxs[j]], bufsb[j], sem_g[j],
                             add=True)
        for j in range(_D):
            c = g * _D + j
            off = base + c * _CH
            pltpu.make_async_copy(ta_hbm.at[idxs[j]], bufsa[j],
                                  sem_g[j]).wait()
            pltpu.make_async_copy(tb_hbm.at[idxs[j]], bufsb[j],
                                  sem_g[j]).wait()
            pltpu.async_copy(bufsa[j],
                             out_hbm.at[pl.ds(off, _CH), pl.ds(0, _HD)],
                             sem_g[j])
            pltpu.async_copy(bufsb[j],
                             out_hbm.at[pl.ds(off, _CH), pl.ds(_HD, _HD)],
                             sem_g[j])
        for j in range(_D):
            c = g * _D + j
            off = base + c * _CH
            pltpu.make_async_copy(
                bufsa[j], out_hbm.at[pl.ds(off, _CH), pl.ds(0, _HD)],
                sem_g[j]).wait()
            pltpu.make_async_copy(
                bufsb[j], out_hbm.at[pl.ds(off, _CH), pl.ds(_HD, _HD)],
                sem_g[j]).wait()
            cn = jnp.minimum((g + 1) * _D + j, _NCHUNK - 1)
            fire_inputs(j, cn)
        return carry

    lax.fori_loop(0, _NG, group, 0)

    for j in range(_D):
        wait_inputs(j, _NCHUNK - 1)


def kernel(x, table, pe):
    pe_rows = pe[0, :_SEQ]
    pe2 = jnp.concatenate([pe_rows, pe_rows[: _PE2 - _SEQ]], axis=0)
    out = _embed_sc(x.reshape(-1).astype(jnp.int32), pe2,
                    table[:, :_HD], table[:, _HD:])
    return out[:, :_NUM_DIM].reshape(_BATCH, _SEQ, _NUM_DIM)


# speedup vs baseline: 1.8617x; 1.8617x over previous
import functools

import jax
import jax.numpy as jnp
from jax import lax
from jax.experimental import pallas as pl
from jax.experimental.pallas import tpu as pltpu
from jax.experimental.pallas import tpu_sc as plsc

_NUM_DIM = 64
_PAD_DIM = 128
_BATCH = 4096
_SEQ = 200

_info = plsc.get_sparse_core_info()
_NC, _NS = _info.num_cores, _info.num_subcores
_NW = _NC * _NS
_TOTAL = _BATCH * _SEQ
_ROWS_PER_W = _TOTAL // _NW
_CH = 128
_NCHUNK = _ROWS_PER_W // _CH
_PE2 = _SEQ + _CH - 8
_D = 8
_NG = _NCHUNK // _D


@functools.partial(
    pl.kernel,
    out_type=jax.ShapeDtypeStruct((_TOTAL, _PAD_DIM), jnp.float32),
    mesh=plsc.VectorSubcoreMesh(core_axis_name="c", subcore_axis_name="s"),
    scratch_types=[
        pltpu.VMEM_SHARED((_PE2, _NUM_DIM), jnp.float32),
        [pltpu.VMEM((_CH,), jnp.int32) for _ in range(_D)],
        [pltpu.VMEM((_CH, _NUM_DIM), jnp.float32) for _ in range(_D)],
        [pltpu.SemaphoreType.DMA for _ in range(_D)],
        [pltpu.SemaphoreType.DMA for _ in range(_D)],
        [pltpu.SemaphoreType.DMA for _ in range(_D)],
    ],
    compiler_params=pltpu.CompilerParams(use_tc_tiling_on_sc=False),
)
def _embed_sc(x_hbm, pe_hbm, table_hbm, out_hbm,
              pe_sh, idxs, bufs, sem_i, sem_p, sem_g):
    wid = lax.axis_index("s") * _NC + lax.axis_index("c")
    base = wid * _ROWS_PER_W

    @pl.when(lax.axis_index("s") == 0)
    def _():
        pltpu.sync_copy(pe_hbm, pe_sh)

    plsc.subcore_barrier()

    def fire_inputs(j, c):
        off = base + c * _CH
        r = (c * _CH) % _SEQ
        pltpu.async_copy(x_hbm.at[pl.ds(off, _CH)], idxs[j], sem_i[j])
        pltpu.async_copy(pe_sh.at[pl.ds(r, _CH)], bufs[j], sem_p[j])

    def wait_inputs(j, c):
        off = base + c * _CH
        r = (c * _CH) % _SEQ
        pltpu.make_async_copy(x_hbm.at[pl.ds(off, _CH)], idxs[j],
                              sem_i[j]).wait()
        pltpu.make_async_copy(pe_sh.at[pl.ds(r, _CH)], bufs[j],
                              sem_p[j]).wait()

    for j in range(_D):
        fire_inputs(j, j)

    def group(g, carry):
        for j in range(_D):
            c = g * _D + j
            wait_inputs(j, c)
            pltpu.async_copy(table_hbm.at[idxs[j]], bufs[j], sem_g[j],
                             add=True)
        for j in range(_D):
            c = g * _D + j
            off = base + c * _CH
            pltpu.make_async_copy(table_hbm.at[idxs[j]], bufs[j],
                                  sem_g[j]).wait()
            pltpu.async_copy(bufs[j],
                             out_hbm.at[pl.ds(off, _CH), pl.ds(0, _NUM_DIM)],
                             sem_g[j])
        for j in range(_D):
            c = g * _D + j
            off = base + c * _CH
            pltpu.make_async_copy(
                bufs[j], out_hbm.at[pl.ds(off, _CH), pl.ds(0, _NUM_DIM)],
                sem_g[j]).wait()
            cn = jnp.minimum((g + 1) * _D + j, _NCHUNK - 1)
            fire_inputs(j, cn)
        return carry

    lax.fori_loop(0, _NG, group, 0)

    for j in range(_D):
        wait_inputs(j, _NCHUNK - 1)


def kernel(x, table, pe):
    pe_rows = pe[0, :_SEQ]
    pe2 = jnp.concatenate([pe_rows, pe_rows[: _PE2 - _SEQ]], axis=0)
    out = _embed_sc(x.reshape(-1).astype(jnp.int32), pe2, table)
    return out[:, :_NUM_DIM].reshape(_BATCH, _SEQ, _NUM_DIM)
